# Initial kernel scaffold; baseline (speedup 1.0000x reference)
import jax
import jax.numpy as jnp
from jax.experimental import pallas as pl


def kernel(feat_word, feat_doc, W_wd, b_wd, W_dw, b_dw, edge_index_wd, edge_index_dw):
    raise NotImplementedError("write your pallas kernel here")


# speedup vs baseline: 3.1856x; 3.1856x over previous
import functools

import jax
import jax.numpy as jnp
from jax import lax
from jax.experimental import pallas as pl
from jax.experimental.pallas import tpu as pltpu
from jax.experimental.pallas import tpu_sc as plsc

N = 10000
D = 128
E = 320000
NCORE = 2
NTILE = 16
NW = NCORE * NTILE
CH = 128
EPW = E // NW
K = -(-EPW // CH)
EPW_PAD = K * CH
PAD_N = 10112
RPT = PAD_N // NTILE


def _mm_body(x_ref, w_ref, b_ref, o_ref):
    o_ref[...] = (
        jnp.dot(x_ref[...], w_ref[0], preferred_element_type=jnp.float32)
        + b_ref[0]
    )


def _matmul(feats, W_all, b_all):
    BM = 2000
    nb = N // BM
    return pl.pallas_call(
        _mm_body,
        grid=(2, nb),
        in_specs=[
            pl.BlockSpec((BM, D), lambda i, j: (i * nb + j, 0)),
            pl.BlockSpec((1, D, D), lambda i, j: (i, 0, 0)),
            pl.BlockSpec((1, 1, D), lambda i, j: (i, 0, 0)),
        ],
        out_specs=pl.BlockSpec((BM, D), lambda i, j: (i * nb + j, 0)),
        out_shape=jax.ShapeDtypeStruct((2 * N, D), jnp.float32),
    )(feats, W_all, b_all)


def _sc_body(wh, src_all, dst_all, zo,
             sums_out,
             src_v, dst_v, rows_v, oidx_v, acc_sh, sem):
    cid = lax.axis_index("c")
    sid = lax.axis_index("s")
    wid = cid * NTILE + sid
    rbase = sid * RPT

    for i in range(CH // 16):
        oidx_v[pl.ds(i * 16, 16)] = jnp.full((16,), PAD_N, jnp.int32)

    for p in range(4):
        e, is_cnt = p // 2, p % 2
        pltpu.sync_copy(zo.at[pl.ds(rbase, RPT)],
                        acc_sh.at[pl.ds(rbase, RPT)])
        if not is_cnt:
            pltpu.sync_copy(src_all.at[e * NW + wid], src_v)
            pltpu.sync_copy(dst_all.at[e * NW + wid], dst_v)
        plsc.subcore_barrier()

        if is_cnt:
            pltpu.async_copy(zo.at[oidx_v], rows_v, sem).wait()

            def cchunk(j, carry):
                pltpu.sync_copy(rows_v, acc_sh.at[dst_v.at[j]], add=True)
                return carry

            lax.fori_loop(0, K, cchunk, 0)
        else:
            def chunk(j, carry):
                pltpu.async_copy(wh.at[src_v.at[j]], rows_v, sem).wait()
                pltpu.sync_copy(rows_v, acc_sh.at[dst_v.at[j]], add=True)
                return carry

            lax.fori_loop(0, K, chunk, 0)

        plsc.subcore_barrier()

        pltpu.sync_copy(acc_sh.at[pl.ds(rbase, RPT)],
                        sums_out.at[p, cid, pl.ds(rbase, RPT)])


_sc_segsum = functools.partial(
    pl.kernel,
    out_type=jax.ShapeDtypeStruct((4, NCORE, PAD_N, D), jnp.float32),
    mesh=plsc.VectorSubcoreMesh(core_axis_name="c", subcore_axis_name="s"),
    scratch_types=[
        pltpu.VMEM((K, CH), jnp.int32),
        pltpu.VMEM((K, CH), jnp.int32),
        pltpu.VMEM((CH, D), jnp.float32),
        pltpu.VMEM((CH,), jnp.int32),
        pltpu.VMEM_SHARED((PAD_N, D), jnp.float32),
        pltpu.SemaphoreType.DMA,
    ],
)(_sc_body)


BF = 2528


def _fin_body(s_ref, c_ref, o_ref):
    s = s_ref[0, 0] + s_ref[0, 1]
    c = c_ref[0, 0][:, 0:1] + c_ref[0, 1][:, 0:1]
    o_ref[0] = jnp.maximum(s / jnp.maximum(c, 1.0), 0.0)


def _finalize(sums):
    nb = PAD_N // BF
    return pl.pallas_call(
        _fin_body,
        grid=(2, nb),
        in_specs=[
            pl.BlockSpec((1, NCORE, BF, D), lambda i, j: (2 * i, 0, j, 0)),
            pl.BlockSpec((1, NCORE, BF, D), lambda i, j: (2 * i + 1, 0, j, 0)),
        ],
        out_specs=pl.BlockSpec((1, BF, D), lambda i, j: (i, j, 0)),
        out_shape=jax.ShapeDtypeStruct((2, PAD_N, D), jnp.float32),
    )(sums, sums)


def _prep_edges(src, dst, src_off):
    src = src.astype(jnp.int32) + src_off
    dst = dst.astype(jnp.int32)
    src = src.reshape(NW, EPW)
    dst = dst.reshape(NW, EPW)
    pad = EPW_PAD - EPW
    src = jnp.pad(src, ((0, 0), (0, pad)), constant_values=0)
    dst = jnp.pad(dst, ((0, 0), (0, pad)), constant_values=N)
    return src.reshape(NW, K, CH), dst.reshape(NW, K, CH)


def kernel(feat_word, feat_doc, W_wd, b_wd, W_dw, b_dw,
           edge_index_wd, edge_index_dw):
    feats = jnp.concatenate([feat_word, feat_doc], axis=0)
    W_all = jnp.stack([W_wd, W_dw])
    b_all = jnp.stack([b_wd, b_dw]).reshape(2, 1, D)
    wh = _matmul(feats, W_all, b_all)

    s_wd, d_wd = _prep_edges(edge_index_wd[0], edge_index_wd[1], 0)
    s_dw, d_dw = _prep_edges(edge_index_dw[0], edge_index_dw[1], N)
    src_all = jnp.concatenate([s_wd, s_dw])
    dst_all = jnp.concatenate([d_wd, d_dw])

    zo = jnp.concatenate([jnp.zeros((PAD_N, D), jnp.float32),
                          jnp.ones((8, D), jnp.float32)])
    sums = _sc_segsum(wh, src_all, dst_all, zo)

    h = _finalize(sums)
    return (h[1, :N], h[0, :N])


# speedup vs baseline: 3.5342x; 1.1094x over previous
import functools

import jax
import jax.numpy as jnp
from jax import lax
from jax.experimental import pallas as pl
from jax.experimental.pallas import tpu as pltpu
from jax.experimental.pallas import tpu_sc as plsc

N = 10000
D = 128
E = 320000
NCORE = 2
NTILE = 16
NW = NCORE * NTILE
CH = 128
EPW = E // NW
K = 80
EPW_PAD = K * CH
PAD_N = 10112
RPT = PAD_N // NTILE
CPAD_N = 10240
CPT = CPAD_N // NTILE


def _mm_body(x_ref, w_ref, b_ref, o_ref):
    o_ref[...] = (
        jnp.dot(x_ref[...], w_ref[0], preferred_element_type=jnp.float32)
        + b_ref[0]
    )


def _matmul(feats, W_all, b_all):
    BM = 2000
    nb = N // BM
    return pl.pallas_call(
        _mm_body,
        grid=(2, nb),
        in_specs=[
            pl.BlockSpec((BM, D), lambda i, j: (i * nb + j, 0)),
            pl.BlockSpec((1, D, D), lambda i, j: (i, 0, 0)),
            pl.BlockSpec((1, 1, D), lambda i, j: (i, 0, 0)),
        ],
        out_specs=pl.BlockSpec((BM, D), lambda i, j: (i * nb + j, 0)),
        out_shape=jax.ShapeDtypeStruct((2 * N, D), jnp.float32),
    )(feats, W_all, b_all)


def _sc_body(wh, src_all, dst_all, zrow,
             sums_out, cnt_out,
             src_v, dst_v, rows_a, ones_v, zv, acc_sh, cnt_sh, sem_a):
    cid = lax.axis_index("c")
    sid = lax.axis_index("s")
    wid = cid * NTILE + sid
    rbase = sid * RPT
    cbase = sid * CPT

    for i in range(CH // 16):
        ones_v[pl.ds(i * 16, 16)] = jnp.full((16,), 1.0, jnp.float32)
    for i in range(CPT // 16):
        zv[pl.ds(i * 16, 16)] = jnp.zeros((16,), jnp.float32)

    for e in range(2):
        pltpu.sync_copy(zrow.at[pl.ds(rbase, RPT)],
                        acc_sh.at[pl.ds(rbase, RPT)])
        pltpu.sync_copy(zv, cnt_sh.at[pl.ds(cbase, CPT)])
        pltpu.sync_copy(src_all.at[e * NW + wid], src_v)
        pltpu.sync_copy(dst_all.at[e * NW + wid], dst_v)
        plsc.subcore_barrier()

        def chunk(j, carry):
            pltpu.async_copy(wh.at[src_v.at[j]], rows_a, sem_a).wait()
            pltpu.sync_copy(rows_a, acc_sh.at[dst_v.at[j]], add=True)
            pltpu.sync_copy(ones_v, cnt_sh.at[dst_v.at[j]], add=True)
            return carry

        lax.fori_loop(0, K, chunk, 0)
        plsc.subcore_barrier()

        pltpu.sync_copy(acc_sh.at[pl.ds(rbase, RPT)],
                        sums_out.at[e, cid, pl.ds(rbase, RPT)])
        pltpu.sync_copy(cnt_sh.at[pl.ds(cbase, CPT)],
                        cnt_out.at[pl.ds((e * NCORE + cid) * CPAD_N + cbase,
                                         CPT)])


_sc_segsum = functools.partial(
    pl.kernel,
    out_type=[
        jax.ShapeDtypeStruct((2, NCORE, PAD_N, D), jnp.float32),
        jax.ShapeDtypeStruct((2 * NCORE * CPAD_N,), jnp.float32),
    ],
    mesh=plsc.VectorSubcoreMesh(core_axis_name="c", subcore_axis_name="s"),
    scratch_types=[
        pltpu.VMEM((K + 2, CH), jnp.int32),
        pltpu.VMEM((K, CH), jnp.int32),
        pltpu.VMEM((CH, D), jnp.float32),
        pltpu.VMEM((CH,), jnp.float32),
        pltpu.VMEM((CPT,), jnp.float32),
        pltpu.VMEM_SHARED((PAD_N, D), jnp.float32),
        pltpu.VMEM_SHARED((CPAD_N,), jnp.float32),
        pltpu.SemaphoreType.DMA,
    ],
)(_sc_body)


def _fin_body(s_ref, c_ref, o_ref):
    s = s_ref[0, 0] + s_ref[0, 1]
    c0 = c_ref[0, 0, :PAD_N]
    c1 = c_ref[0, 0, CPAD_N:CPAD_N + PAD_N]
    c = (c0 + c1).reshape(PAD_N, 1)
    o_ref[0] = jnp.maximum(s / jnp.maximum(c, 1.0), 0.0)


def _finalize(sums, cnts):
    return pl.pallas_call(
        _fin_body,
        grid=(2,),
        in_specs=[
            pl.BlockSpec((1, NCORE, PAD_N, D), lambda i: (i, 0, 0, 0)),
            pl.BlockSpec((1, 1, NCORE * CPAD_N), lambda i: (i, 0, 0)),
        ],
        out_specs=pl.BlockSpec((1, PAD_N, D), lambda i: (i, 0, 0)),
        out_shape=jax.ShapeDtypeStruct((2, PAD_N, D), jnp.float32),
    )(sums, cnts)


def _prep_edges(src, dst, src_off):
    src = src.astype(jnp.int32) + src_off
    dst = dst.astype(jnp.int32)
    src = src.reshape(NW, EPW)
    dst = dst.reshape(NW, EPW)
    pad = EPW_PAD - EPW
    src = jnp.pad(src, ((0, 0), (0, pad + 2 * CH)), constant_values=0)
    dst = jnp.pad(dst, ((0, 0), (0, pad)), constant_values=N)
    return src.reshape(NW, K + 2, CH), dst.reshape(NW, K, CH)


def kernel(feat_word, feat_doc, W_wd, b_wd, W_dw, b_dw,
           edge_index_wd, edge_index_dw):
    feats = jnp.concatenate([feat_word, feat_doc], axis=0)
    W_all = jnp.stack([W_wd, W_dw])
    b_all = jnp.stack([b_wd, b_dw]).reshape(2, 1, D)
    wh = _matmul(feats, W_all, b_all)

    s_wd, d_wd = _prep_edges(edge_index_wd[0], edge_index_wd[1], 0)
    s_dw, d_dw = _prep_edges(edge_index_dw[0], edge_index_dw[1], N)
    src_all = jnp.concatenate([s_wd, s_dw])
    dst_all = jnp.concatenate([d_wd, d_dw])

    zrow = jnp.zeros((PAD_N, D), jnp.float32)
    sums, cnts = _sc_segsum(wh, src_all, dst_all, zrow)

    h = _finalize(sums, cnts.reshape(2, 1, NCORE * CPAD_N))
    return (h[1, :N], h[0, :N])


# speedup vs baseline: 5.0701x; 1.4346x over previous
import functools

import jax
import jax.numpy as jnp
from jax import lax
from jax.experimental import pallas as pl
from jax.experimental.pallas import tpu as pltpu
from jax.experimental.pallas import tpu_sc as plsc

N = 10000
D = 128
E = 320000
NCORE = 2
NTILE = 16
NW = NCORE * NTILE
CH = 128
EPW = E // NW
K = -(-EPW // CH)
EPW_PAD = K * CH
PAD_N = 10112
RPT = PAD_N // NTILE
CPAD_N = 10240
CPT = CPAD_N // NTILE


def _mm_body(x_ref, w_ref, b_ref, o_ref):
    o_ref[...] = (
        jnp.dot(x_ref[...], w_ref[0], preferred_element_type=jnp.float32)
        + b_ref[0]
    )


def _matmul(feats, W_all, b_all):
    BM = 2000
    nb = N // BM
    return pl.pallas_call(
        _mm_body,
        grid=(2, nb),
        in_specs=[
            pl.BlockSpec((BM, D), lambda i, j: (i * nb + j, 0)),
            pl.BlockSpec((1, D, D), lambda i, j: (i, 0, 0)),
            pl.BlockSpec((1, 1, D), lambda i, j: (i, 0, 0)),
        ],
        out_specs=pl.BlockSpec((BM, D), lambda i, j: (i * nb + j, 0)),
        out_shape=jax.ShapeDtypeStruct((2 * N, D), jnp.float32),
    )(feats, W_all, b_all)


def _sc_body(wh, src_all, dst_all, zrow,
             sums_out, cnt_out,
             src_v, dst_v, rows_v, ones_v, zv, acc_sh, cnt_sh, sem):
    cid = lax.axis_index("c")
    sid = lax.axis_index("s")
    wid = cid * NTILE + sid
    rbase = sid * RPT
    cbase = sid * CPT

    for i in range(CH // 16):
        ones_v[pl.ds(i * 16, 16)] = jnp.full((16,), 1.0, jnp.float32)
    for i in range(CPT // 16):
        zv[pl.ds(i * 16, 16)] = jnp.zeros((16,), jnp.float32)

    for e in range(2):
        pltpu.sync_copy(zrow.at[pl.ds(rbase, RPT)],
                        acc_sh.at[pl.ds(rbase, RPT)])
        pltpu.sync_copy(zv, cnt_sh.at[pl.ds(cbase, CPT)])
        pltpu.sync_copy(src_all.at[e * NW + wid], src_v)
        pltpu.sync_copy(dst_all.at[e * NW + wid], dst_v)
        plsc.subcore_barrier()

        def chunk(j, carry):
            pltpu.async_copy(wh.at[src_v.at[j]], rows_v, sem).wait()
            pltpu.sync_copy(rows_v, acc_sh.at[dst_v.at[j]], add=True)
            pltpu.sync_copy(ones_v, cnt_sh.at[dst_v.at[j]], add=True)
            return carry

        lax.fori_loop(0, K, chunk, 0)
        plsc.subcore_barrier()

        pltpu.sync_copy(acc_sh.at[pl.ds(rbase, RPT)],
                        sums_out.at[e, cid, pl.ds(rbase, RPT)])
        pltpu.sync_copy(cnt_sh.at[pl.ds(cbase, CPT)],
                        cnt_out.at[pl.ds((e * NCORE + cid) * CPAD_N + cbase,
                                         CPT)])


_sc_segsum = functools.partial(
    pl.kernel,
    out_type=[
        jax.ShapeDtypeStruct((2, NCORE, PAD_N, D), jnp.float32),
        jax.ShapeDtypeStruct((2 * NCORE * CPAD_N,), jnp.float32),
    ],
    mesh=plsc.VectorSubcoreMesh(core_axis_name="c", subcore_axis_name="s"),
    scratch_types=[
        pltpu.VMEM((K, CH), jnp.int32),
        pltpu.VMEM((K, CH), jnp.int32),
        pltpu.VMEM((CH, D), jnp.float32),
        pltpu.VMEM((CH,), jnp.float32),
        pltpu.VMEM((CPT,), jnp.float32),
        pltpu.VMEM_SHARED((PAD_N, D), jnp.float32),
        pltpu.VMEM_SHARED((CPAD_N,), jnp.float32),
        pltpu.SemaphoreType.DMA,
    ],
)(_sc_body)


def _fin_body(s_ref, c_ref, o_ref):
    s = s_ref[0, 0] + s_ref[0, 1]
    c0 = c_ref[0, 0, :PAD_N]
    c1 = c_ref[0, 0, CPAD_N:CPAD_N + PAD_N]
    c = (c0 + c1).reshape(PAD_N, 1)
    o_ref[0] = jnp.maximum(s / jnp.maximum(c, 1.0), 0.0)


def _finalize(sums, cnts):
    return pl.pallas_call(
        _fin_body,
        grid=(2,),
        in_specs=[
            pl.BlockSpec((1, NCORE, PAD_N, D), lambda i: (i, 0, 0, 0)),
            pl.BlockSpec((1, 1, NCORE * CPAD_N), lambda i: (i, 0, 0)),
        ],
        out_specs=pl.BlockSpec((1, PAD_N, D), lambda i: (i, 0, 0)),
        out_shape=jax.ShapeDtypeStruct((2, PAD_N, D), jnp.float32),
    )(sums, cnts)


def _prep_edges(src, dst, src_off):
    src = src.astype(jnp.int32) + src_off
    dst = dst.astype(jnp.int32)
    src = src.reshape(NW, EPW)
    dst = dst.reshape(NW, EPW)
    pad = EPW_PAD - EPW
    src = jnp.pad(src, ((0, 0), (0, pad)), constant_values=0)
    dst = jnp.pad(dst, ((0, 0), (0, pad)), constant_values=N)
    return src.reshape(NW, K, CH), dst.reshape(NW, K, CH)


def kernel(feat_word, feat_doc, W_wd, b_wd, W_dw, b_dw,
           edge_index_wd, edge_index_dw):
    feats = jnp.concatenate([feat_word, feat_doc], axis=0)
    W_all = jnp.stack([W_wd, W_dw])
    b_all = jnp.stack([b_wd, b_dw]).reshape(2, 1, D)
    wh = _matmul(feats, W_all, b_all)

    s_wd, d_wd = _prep_edges(edge_index_wd[0], edge_index_wd[1], 0)
    s_dw, d_dw = _prep_edges(edge_index_dw[0], edge_index_dw[1], N)
    src_all = jnp.concatenate([s_wd, s_dw])
    dst_all = jnp.concatenate([d_wd, d_dw])

    zrow = jnp.zeros((PAD_N, D), jnp.float32)
    sums, cnts = _sc_segsum(wh, src_all, dst_all, zrow)

    h = _finalize(sums, cnts.reshape(2, 1, NCORE * CPAD_N))
    return (h[1, :N], h[0, :N])
